# baseline (speedup 1.0000x reference)
import functools

import jax
import jax.numpy as jnp
from jax import lax
from jax.experimental import pallas as pl
from jax.experimental.pallas import tpu as pltpu
from jax.experimental.pallas import tpu_sc as plsc

N = 100000
E = 1600000
H = 128
HB = 64
G = 512

NC = 2
NS = 16
NW = NC * NS
CHUNK = 12800
NCHUNK = 8
TRASH = CHUNK
ACC_ROWS = CHUNK + 16
ROWW = 128
CAP_ROWS = 392
EPT = E // NW
BLK = 2000
NBLK = EPT // BLK
VECS = BLK // 16
NPAD = 102400
SH = CHUNK // NS
SH7A = 656
SH7B = (N - 7 * CHUNK) - (NS - 1) * SH7A
LISTW = NCHUNK * CAP_ROWS * ROWW

_mesh = plsc.VectorSubcoreMesh(core_axis_name="c", subcore_axis_name="s",
                               num_cores=NC, num_subcores=NS)
_sc_params = pltpu.CompilerParams(needs_layout_passes=False)


@functools.partial(
    pl.kernel,
    out_type=[
        jax.ShapeDtypeStruct((NW * LISTW,), jnp.int32),
        jax.ShapeDtypeStruct((NW * LISTW,), jnp.int32),
        jax.ShapeDtypeStruct((NW * 16,), jnp.int32),
        jax.ShapeDtypeStruct((NW * N,), jnp.float32),
    ],
    mesh=_mesh,
    compiler_params=_sc_params,
    scratch_types=[
        pltpu.VMEM((BLK,), jnp.int32),
        pltpu.VMEM((BLK,), jnp.int32),
        pltpu.VMEM((N,), jnp.float32),
        *[pltpu.VMEM((288,), jnp.int32) for _ in range(2 * NCHUNK)],
        pltpu.VMEM((16,), jnp.int32),
    ],
)
def _prep(src_hbm, dst_hbm, bsrc_hbm, bdst_hbm, rowcnt_hbm, degp_hbm,
          srcblk, dstblk, hist, *rest):
    bufs = rest[:NCHUNK]
    bufd = rest[NCHUNK:2 * NCHUNK]
    cntv = rest[2 * NCHUNK]
    wid = lax.axis_index("c") * NS + lax.axis_index("s")
    zf = jnp.zeros((16,), jnp.float32)
    zi = jnp.zeros((16,), jnp.int32)
    ones = jnp.ones((16,), jnp.float32)
    trash = jnp.full((16,), TRASH, jnp.int32)

    def zero_hist(i, _):
        hist[pl.ds(i * 16, 16)] = zf
        return 0
    lax.fori_loop(0, N // 16, zero_hist, 0)

    def row_out(b, r):
        return wid * LISTW + (b * CAP_ROWS + r) * ROWW

    def block_body(blk, carry):
        base_e = wid * EPT + blk * BLK
        pltpu.sync_copy(src_hbm.at[pl.ds(base_e, BLK)], srcblk)
        pltpu.sync_copy(dst_hbm.at[pl.ds(base_e, BLK)], dstblk)

        def vec_body(j, st):
            cnts, rps = st
            d = dstblk[pl.ds(j * 16, 16)]
            s = srcblk[pl.ds(j * 16, 16)]
            plsc.addupdate_scatter(hist, [d], ones)
            bid = jnp.zeros((16,), jnp.int32)
            for k in range(1, NCHUNK):
                bid = bid + (d >= k * CHUNK).astype(jnp.int32)
            new_cnts = []
            new_rps = []
            for b in range(NCHUNK):
                m = bid == b
                cb = cnts[b]
                plsc.store_compressed(bufs[b].at[pl.ds(cb, 16)], s, mask=m)
                plsc.store_compressed(bufd[b].at[pl.ds(cb, 16)],
                                      d - b * CHUNK, mask=m)
                cb = cb + jnp.sum(m.astype(jnp.int32))
                full = cb >= ROWW
                rb = rps[b]

                @pl.when(full)
                def _():
                    pltpu.sync_copy(bufs[b].at[pl.ds(0, ROWW)],
                                    bsrc_hbm.at[pl.ds(row_out(b, rb), ROWW)])
                    pltpu.sync_copy(bufd[b].at[pl.ds(0, ROWW)],
                                    bdst_hbm.at[pl.ds(row_out(b, rb), ROWW)])
                    vs = bufs[b][pl.ds(ROWW, 16)]
                    vd = bufd[b][pl.ds(ROWW, 16)]
                    bufs[b][pl.ds(0, 16)] = vs
                    bufd[b][pl.ds(0, 16)] = vd
                new_cnts.append(jnp.where(full, cb - ROWW, cb))
                new_rps.append(jnp.where(full, rb + 1, rb))
            return (tuple(new_cnts), tuple(new_rps))

        return lax.fori_loop(0, VECS, vec_body, carry)

    z = jnp.int32(0)
    cnts, rps = lax.fori_loop(0, NBLK, block_body,
                              ((z,) * NCHUNK, (z,) * NCHUNK))

    iota = lax.iota(jnp.int32, 16)
    rowcnt_vec = jnp.zeros((16,), jnp.int32)
    for b in range(NCHUNK):
        cb, rb = cnts[b], rps[b]

        @pl.when(cb > 0)
        def _():
            for k in range(8):
                plsc.store_compressed(bufs[b].at[pl.ds(cb + k * 16, 16)], zi,
                                      mask=iota >= 0)
                plsc.store_compressed(bufd[b].at[pl.ds(cb + k * 16, 16)],
                                      trash, mask=iota >= 0)
            pltpu.sync_copy(bufs[b].at[pl.ds(0, ROWW)],
                            bsrc_hbm.at[pl.ds(row_out(b, rb), ROWW)])
            pltpu.sync_copy(bufd[b].at[pl.ds(0, ROWW)],
                            bdst_hbm.at[pl.ds(row_out(b, rb), ROWW)])
        rb2 = rb + (cb > 0).astype(jnp.int32)
        for k in range(8):
            bufs[b][pl.ds(k * 16, 16)] = zi
            bufd[b][pl.ds(k * 16, 16)] = trash
        tgt = ((rb2 + 7) >> 3) << 3
        for k in range(7):
            @pl.when(rb2 + k < tgt)
            def _():
                pltpu.sync_copy(bufs[b].at[pl.ds(0, ROWW)],
                                bsrc_hbm.at[pl.ds(row_out(b, rb2 + k), ROWW)])
                pltpu.sync_copy(bufd[b].at[pl.ds(0, ROWW)],
                                bdst_hbm.at[pl.ds(row_out(b, rb2 + k), ROWW)])
        rowcnt_vec = jnp.where(iota == b, tgt, rowcnt_vec)

    cntv[...] = rowcnt_vec
    pltpu.sync_copy(cntv, rowcnt_hbm.at[pl.ds(wid * 16, 16)])
    pltpu.sync_copy(hist, degp_hbm.at[pl.ds(wid * N, N)])


@functools.partial(
    pl.kernel,
    out_type=jax.ShapeDtypeStruct((N, H), jnp.float32),
    mesh=_mesh,
    compiler_params=_sc_params,
    scratch_types=[
        pltpu.VMEM((8 * ROWW,), jnp.int32),
        pltpu.VMEM((8 * ROWW,), jnp.int32),
        pltpu.VMEM((8, ROWW), jnp.int32),
        pltpu.VMEM((ROWW, H), jnp.float32),
        pltpu.VMEM((NW * 16,), jnp.int32),
        pltpu.VMEM_SHARED((ACC_ROWS, H), jnp.float32),
        pltpu.SemaphoreType.DMA,
    ],
)
def _conv(hp_hbm, bsrc_hbm, bdst_hbm, rowcnt_hbm, out_hbm,
          slin, dlin, dbuf, rows, rcv, acc, sem):
    cid = lax.axis_index("c")
    tid = lax.axis_index("s")
    zf = jnp.zeros((16,), jnp.float32)
    iota = lax.iota(jnp.int32, 16)

    pltpu.sync_copy(rowcnt_hbm, rcv)

    for p in range(NCHUNK // NC):
        chunk = p * NC + cid
        base = chunk * CHUNK

        def zrow(i, _):
            for k in range(H // 16):
                rows[i, pl.ds(k * 16, 16)] = zf
            return 0
        lax.fori_loop(0, ROWW, zrow, 0)
        for k in range(SH // ROWW):
            pltpu.sync_copy(rows, acc.at[pl.ds(tid * SH + k * ROWW, ROWW)])
        pltpu.sync_copy(rows.at[pl.ds(0, SH % ROWW)],
                        acc.at[pl.ds(tid * SH + (SH // ROWW) * ROWW,
                                     SH % ROWW)])
        plsc.subcore_barrier()

        for li in range(2):
            l = tid * 2 + li
            rvec = rcv[pl.ds(l * 16, 16)]
            nblk = jnp.sum(jnp.where(iota == chunk, rvec, 0)) >> 3

            def blk_body(bi, _):
                fb = l * LISTW + (chunk * CAP_ROWS + bi * 8) * ROWW
                pltpu.sync_copy(bsrc_hbm.at[pl.ds(fb, 8 * ROWW)], slin)
                pltpu.sync_copy(bdst_hbm.at[pl.ds(fb, 8 * ROWW)], dlin)
                for r in range(8):
                    for c in range(ROWW // 16):
                        dbuf[r, pl.ds(c * 16, 16)] = dlin[
                            pl.ds(r * ROWW + c * 16, 16)]
                for j in range(8):
                    pltpu.async_copy(
                        hp_hbm.at[slin.at[pl.ds(j * ROWW, ROWW)]],
                        rows, sem).wait()
                    pltpu.sync_copy(rows, acc.at[dbuf.at[j]], add=True)
                return 0

            lax.fori_loop(0, nblk, blk_body, 0)

        plsc.subcore_barrier()

        @pl.when(base + CHUNK <= N)
        def _():
            pltpu.sync_copy(acc.at[pl.ds(tid * SH, SH)],
                            out_hbm.at[pl.ds(base + tid * SH, SH)])

        @pl.when((base + CHUNK > N) & (tid < NS - 1))
        def _():
            pltpu.sync_copy(acc.at[pl.ds(tid * SH7A, SH7A)],
                            out_hbm.at[pl.ds(base + tid * SH7A, SH7A)])

        @pl.when((base + CHUNK > N) & (tid == NS - 1))
        def _():
            pltpu.sync_copy(
                acc.at[pl.ds((NS - 1) * SH7A, SH7B)],
                out_hbm.at[pl.ds(base + (NS - 1) * SH7A, SH7B)])
        plsc.subcore_barrier()


RPT = NPAD // (NW * ROWW)


@functools.partial(
    pl.kernel,
    out_type=[
        jax.ShapeDtypeStruct((NC, G, H), jnp.float32),
        jax.ShapeDtypeStruct((NW * G,), jnp.float32),
    ],
    mesh=_mesh,
    compiler_params=_sc_params,
    scratch_types=[
        pltpu.VMEM((RPT * ROWW,), jnp.int32),
        pltpu.VMEM((RPT, ROWW), jnp.int32),
        pltpu.VMEM((ROWW, H), jnp.float32),
        pltpu.VMEM((G + 16,), jnp.float32),
        pltpu.VMEM_SHARED((G, H), jnp.float32),
    ],
)
def _pool(h3_hbm, b2d_hbm, pacc_hbm, pcnt_hbm, blin, bidx, rows, cnt, acc):
    cid = lax.axis_index("c")
    tid = lax.axis_index("s")
    wid = cid * NS + tid
    zf = jnp.zeros((16,), jnp.float32)
    iota = lax.iota(jnp.int32, 16)

    def zcnt(i, _):
        cnt[pl.ds(i * 16, 16)] = zf
        return 0
    lax.fori_loop(0, (G + 16) // 16, zcnt, 0)

    def zrow(i, _):
        for k in range(H // 16):
            rows[i, pl.ds(k * 16, 16)] = zf
        return 0
    lax.fori_loop(0, ROWW, zrow, 0)
    pltpu.sync_copy(rows.at[pl.ds(0, G // NS)],
                    acc.at[pl.ds(tid * (G // NS), G // NS)])
    plsc.subcore_barrier()

    pltpu.sync_copy(b2d_hbm.at[pl.ds(wid * RPT * ROWW, RPT * ROWW)], blin)
    for r in range(RPT):
        for c in range(ROWW // 16):
            bidx[r, pl.ds(c * 16, 16)] = blin[pl.ds(r * ROWW + c * 16, 16)]
    for r in range(RPT):
        pltpu.sync_copy(h3_hbm.at[pl.ds(wid * RPT * ROWW + r * ROWW, ROWW)],
                        rows)
        pltpu.sync_copy(rows, acc.at[bidx.at[r]], add=True)
        for k in range(ROWW // 16):
            bv = bidx[r, pl.ds(k * 16, 16)]
            gid = wid * RPT * ROWW + r * ROWW + k * 16 + iota
            val = jnp.where(gid < N, 1.0, 0.0).astype(jnp.float32)
            plsc.addupdate_scatter(cnt, [bv], val)
    plsc.subcore_barrier()

    pltpu.sync_copy(acc.at[pl.ds(tid * (G // NS), G // NS)],
                    pacc_hbm.at[cid].at[pl.ds(tid * (G // NS), G // NS)])
    pltpu.sync_copy(cnt.at[pl.ds(0, G)], pcnt_hbm.at[pl.ds(wid * G, G)])


RB = 2000
NRB = N // RB


def _dinv_body(degp_ref, dinv_ref):
    deg = jnp.sum(degp_ref[...], axis=1, keepdims=True) + 1.0
    dinv_ref[...] = lax.rsqrt(deg)


def _tc_dinv(degp_t):
    return pl.pallas_call(
        _dinv_body,
        grid=(NRB,),
        in_specs=[pl.BlockSpec((RB, NW), lambda i: (i, 0))],
        out_specs=pl.BlockSpec((RB, 1), lambda i: (i, 0)),
        out_shape=jax.ShapeDtypeStruct((N, 1), jnp.float32),
    )(degp_t)


def _pre_body(x_ref, w_ref, dinv_ref, hp_ref):
    hp_ref[...] = dinv_ref[...] * jnp.dot(
        x_ref[...], w_ref[...], preferred_element_type=jnp.float32)


def _tc_pre(x, W1, dinv):
    return pl.pallas_call(
        _pre_body,
        grid=(NRB,),
        in_specs=[
            pl.BlockSpec((RB, 32), lambda i: (i, 0)),
            pl.BlockSpec((32, H), lambda i: (0, 0)),
            pl.BlockSpec((RB, 1), lambda i: (i, 0)),
        ],
        out_specs=pl.BlockSpec((RB, H), lambda i: (i, 0)),
        out_shape=jax.ShapeDtypeStruct((N, H), jnp.float32),
    )(x, W1, dinv)


def _stats_body(s_ref, hp_ref, dinv_ref, b_ref, c_ref, st_ref):
    i = pl.program_id(0)
    c = dinv_ref[...] * (s_ref[...] + hp_ref[...]) + b_ref[...]
    c_ref[...] = c
    part = jnp.concatenate(
        [jnp.sum(c, axis=0, keepdims=True),
         jnp.sum(c * c, axis=0, keepdims=True)], axis=0)

    @pl.when(i == 0)
    def _():
        st_ref[...] = part

    @pl.when(i > 0)
    def _():
        st_ref[...] = st_ref[...] + part


def _tc_stats(S, hp, dinv, b):
    return pl.pallas_call(
        _stats_body,
        grid=(NRB,),
        in_specs=[
            pl.BlockSpec((RB, H), lambda i: (i, 0)),
            pl.BlockSpec((RB, H), lambda i: (i, 0)),
            pl.BlockSpec((RB, 1), lambda i: (i, 0)),
            pl.BlockSpec((1, H), lambda i: (0, 0)),
        ],
        out_specs=[
            pl.BlockSpec((RB, H), lambda i: (i, 0)),
            pl.BlockSpec((2, H), lambda i: (0, 0)),
        ],
        out_shape=[
            jax.ShapeDtypeStruct((NPAD, H), jnp.float32),
            jax.ShapeDtypeStruct((2, H), jnp.float32),
        ],
    )(S, hp, dinv, b.reshape(1, H))


def _bn_relu_block(c, st, g, be, eps=1e-5):
    m = st[0:1] * (1.0 / N)
    v = st[1:2] * (1.0 / N) - m * m
    return jnp.maximum(g * (c - m) * lax.rsqrt(v + eps) + be, 0.0)


def _apply_body(c_ref, st_ref, g_ref, be_ref, dinv_ref, w_ref, hp_ref):
    h = _bn_relu_block(c_ref[...], st_ref[...], g_ref[...], be_ref[...])
    hp_ref[...] = dinv_ref[...] * jnp.dot(
        h, w_ref[...], preferred_element_type=jnp.float32)


def _tc_apply(c, st, g, be, dinv, Wn):
    return pl.pallas_call(
        _apply_body,
        grid=(NRB,),
        in_specs=[
            pl.BlockSpec((RB, H), lambda i: (i, 0)),
            pl.BlockSpec((2, H), lambda i: (0, 0)),
            pl.BlockSpec((1, H), lambda i: (0, 0)),
            pl.BlockSpec((1, H), lambda i: (0, 0)),
            pl.BlockSpec((RB, 1), lambda i: (i, 0)),
            pl.BlockSpec((H, H), lambda i: (0, 0)),
        ],
        out_specs=pl.BlockSpec((RB, H), lambda i: (i, 0)),
        out_shape=jax.ShapeDtypeStruct((N, H), jnp.float32),
    )(c, st, g.reshape(1, H), be.reshape(1, H), dinv, Wn)


PB = 2048
NPB = NPAD // PB


def _apply3_body(c_ref, st_ref, g_ref, be_ref, h3_ref):
    i = pl.program_id(0)
    h = _bn_relu_block(c_ref[...], st_ref[...], g_ref[...], be_ref[...])
    rid = i * PB + lax.broadcasted_iota(jnp.int32, (PB, H), 0)
    h3_ref[...] = jnp.where(rid < N, h, 0.0)


def _tc_apply3(c, st, g, be):
    return pl.pallas_call(
        _apply3_body,
        grid=(NPB,),
        in_specs=[
            pl.BlockSpec((PB, H), lambda i: (i, 0)),
            pl.BlockSpec((2, H), lambda i: (0, 0)),
            pl.BlockSpec((1, H), lambda i: (0, 0)),
            pl.BlockSpec((1, H), lambda i: (0, 0)),
        ],
        out_specs=pl.BlockSpec((PB, H), lambda i: (i, 0)),
        out_shape=jax.ShapeDtypeStruct((NPAD, H), jnp.float32),
    )(c, st, g.reshape(1, H), be.reshape(1, H))


def _heads_body(pacc_ref, cnt_ref, wb_ref, bb_ref, wh_ref, bh_ref, out_ref):
    p = pacc_ref[0:G] + pacc_ref[G:2 * G]
    c = jnp.sum(cnt_ref[...], axis=1, keepdims=True)
    pooled = p / jnp.maximum(c, 1.0)
    hb = jnp.maximum(jnp.dot(pooled, wb_ref[...],
                             preferred_element_type=jnp.float32)
                     + bb_ref[...], 0.0)
    out_ref[...] = jnp.dot(hb, wh_ref[...],
                           preferred_element_type=jnp.float32) + bh_ref[...]


def _tc_heads(pacc2, pcnt_t, Wb, bb, Wh, bh):
    return pl.pallas_call(
        _heads_body,
        out_shape=jax.ShapeDtypeStruct((G, 4), jnp.float32),
    )(pacc2, pcnt_t, Wb, bb.reshape(1, HB), Wh, bh.reshape(1, 4))


def kernel(x, edge_index, batch, W1, b1, g1, be1, W2, b2, g2, be2, W3, b3,
           g3, be3, Wb, bb, Wha, bha, Whb, bhb, Whd, bhd, Whg, bhg):
    src, dst = edge_index[0], edge_index[1]
    bsrc, bdst, rowcnt, degp = _prep(src, dst)
    dinv = _tc_dinv(jnp.transpose(degp.reshape(NW, N)))

    hp1 = _tc_pre(x, W1, dinv)
    S1 = _conv(hp1, bsrc, bdst, rowcnt)
    c1, st1 = _tc_stats(S1, hp1, dinv, b1)
    def _japply(c, st, g, be, W):
        m = st[0:1] / N
        v = st[1:2] / N - m * m
        h = jnp.maximum(g * (c - m) * jax.lax.rsqrt(v + 1e-5) + be, 0.0)
        return dinv * (h @ W)
    hp2 = _tc_apply(c1, st1, g1, be1, dinv, W2)

    S2 = _conv(hp2, bsrc, bdst, rowcnt)
    c2, st2 = _tc_stats(S2, hp2, dinv, b2)
    hp3 = _tc_apply(c2, st2, g2, be2, dinv, W3)

    S3 = _conv(hp3, bsrc, bdst, rowcnt)
    c3, st3 = _tc_stats(S3, hp3, dinv, b3)
    h3p = _tc_apply3(c3, st3, g3, be3)

    b2d = jnp.concatenate([batch, jnp.zeros((NPAD - N,), jnp.int32)])
    pacc, pcnt = _pool(h3p, b2d)

    Wh = jnp.concatenate([Wha, Whb, Whd, Whg], axis=1)
    bh = jnp.concatenate([bha, bhb, bhd, bhg], axis=0)
    return _tc_heads(pacc.reshape(NC * G, H),
                     jnp.transpose(pcnt.reshape(NW, G)), Wb, bb, Wh, bh)
